# baseline (speedup 1.0000x reference)
import functools

import jax
import jax.numpy as jnp
from jax import lax
from jax.experimental import pallas as pl
from jax.experimental.pallas import tpu as pltpu
from jax.experimental.pallas import tpu_sc as plsc

B = 16384
D = 16
NT = 14
F1 = 128

NC = 2
NS = 16
NW = NC * NS
BPW = B // NW


@functools.lru_cache(maxsize=None)
def _make_detile(V):
    dbk = min(32768, ((V + 127) // 128) * 128)
    nb = (V + dbk - 1) // dbk
    NB = (V + 127) // 128

    def body(in_ref, out_ref):
        x = in_ref[...]
        out_ref[...] = x.reshape(D, dbk // 128, 128).transpose(1, 0, 2)

    return pl.pallas_call(
        body,
        grid=(nb,),
        in_specs=[pl.BlockSpec((D, dbk), lambda i: (0, i))],
        out_specs=pl.BlockSpec((dbk // 128, D, 128), lambda i: (i, 0, 0)),
        out_shape=jax.ShapeDtypeStruct((NB, D, 128), jnp.float32),
    )


_sc_mesh = plsc.VectorSubcoreMesh(core_axis_name="c", subcore_axis_name="s")

_CHUNKS = ((0, 1, 2, 3, 4), (5, 6, 7, 8, 9), (10, 11, 12, 13))
_CW = 5


@functools.partial(
    pl.kernel,
    out_type=jax.ShapeDtypeStruct((NT, D, B), jnp.float32),
    mesh=_sc_mesh,
    scratch_types=[
        pltpu.VMEM((NT, BPW), jnp.int32),
        pltpu.VMEM((2, _CW, D, BPW), jnp.int32),
        pltpu.VMEM((_CW, D, BPW), jnp.float32),
        pltpu.SemaphoreType.DMA,
        pltpu.SemaphoreType.DMA,
    ],
    compiler_params=pltpu.CompilerParams(use_tc_tiling_on_sc=False),
)
def _sc_gather(*refs):
    tables = refs[:NT]
    idxs = refs[NT:2 * NT]
    out_hbm = refs[2 * NT]
    idx_v, addr_v, rows_v, sem, sem2 = refs[2 * NT + 1:]

    wid = lax.axis_index("s") * NC + lax.axis_index("c")
    base = wid * BPW

    idx_copies = [
        pltpu.async_copy(idxs[t].at[pl.ds(base, BPW)], idx_v.at[t], sem)
        for t in range(NT)
    ]
    for c in idx_copies:
        c.wait()

    def build_addr(ts, slot):
        def addr_body(g, carry):
            for j, t in enumerate(ts):
                vr = idx_v[t, pl.ds(g * 16, 16)]
                vhi = ((vr >> 7) << 11) + (vr & 127)
                for c in range(D):
                    addr_v[slot, j, c, pl.ds(g * 16, 16)] = vhi + c * 128
            return carry

        lax.fori_loop(0, BPW // 16, addr_body, 0)

    def fire_gathers(ts, slot):
        return [
            pltpu.async_copy(
                tables[t].at[addr_v.at[slot, j, c]],
                rows_v.at[j, c], sem)
            for j, t in enumerate(ts)
            for c in range(D)
        ]

    def fire_writes(ts):
        return [
            pltpu.async_copy(
                rows_v.at[j], out_hbm.at[t, :, pl.ds(base, BPW)], sem2)
            for j, t in enumerate(ts)
        ]

    build_addr(_CHUNKS[0], 0)
    gat = fire_gathers(_CHUNKS[0], 0)
    for k in range(1, len(_CHUNKS) + 1):
        if k < len(_CHUNKS):
            build_addr(_CHUNKS[k], k % 2)
        for c in gat:
            c.wait()
        wr = fire_writes(_CHUNKS[k - 1])
        for c in wr:
            c.wait()
        if k < len(_CHUNKS):
            gat = fire_gathers(_CHUNKS[k], k % 2)


BLK = 2048


def _mlp_body(emb_ref, price_ref, w1t_ref, w1p_ref, b1_ref, w2_ref, b2_ref,
              out_ref):
    pT = price_ref[...]
    fc1 = lax.dot_general(
        pT, w1p_ref[...], (((0,), (0,)), ((), ())),
        preferred_element_type=jnp.float32,
        precision=lax.Precision.DEFAULT) + b1_ref[...]
    for t in range(NT):
        fc1 = fc1 + lax.dot_general(
            emb_ref[t], w1t_ref[t], (((0,), (0,)), ((), ())),
            preferred_element_type=jnp.float32,
            precision=lax.Precision.DEFAULT)
    fc1 = jnp.maximum(fc1, 0.0)
    zT = lax.dot_general(
        w2_ref[...], fc1, (((1,), (1,)), ((), ())),
        preferred_element_type=jnp.float32,
        precision=lax.Precision.DEFAULT)
    out_ref[...] = 1.0 / (1.0 + jnp.exp(-(zT + b2_ref[...])))


_mlp = pl.pallas_call(
    _mlp_body,
    grid=(B // BLK,),
    in_specs=[
        pl.BlockSpec((NT, D, BLK), lambda i: (0, 0, i)),
        pl.BlockSpec((1, BLK), lambda i: (0, i)),
        pl.BlockSpec((NT, D, F1), lambda i: (0, 0, 0)),
        pl.BlockSpec((1, F1), lambda i: (0, 0)),
        pl.BlockSpec((1, F1), lambda i: (0, 0)),
        pl.BlockSpec((1, F1), lambda i: (0, 0)),
        pl.BlockSpec((1, 1), lambda i: (0, 0)),
    ],
    out_specs=pl.BlockSpec((1, BLK), lambda i: (0, i)),
    out_shape=jax.ShapeDtypeStruct((1, B), jnp.float32),
)


def kernel(userId, cmsSegId, cmsGroupId, finalGenderCode, ageLevel,
           pvalueLevel, shoppingLevel, occupation, newUserClassLevel,
           adGroupId, cateId, campaignId, customer, brand, price,
           userId_table, cmsSegId_table, cmsGroupId_table,
           finalGenderCode_table, ageLevel_table, pvalueLevel_table,
           shoppingLevel_table, occupation_table, newUserClassLevel_table,
           adGroupId_table, cateId_table, campaignId_table, customer_table,
           brand_table, W1, b1, W2, b2):
    tables = (userId_table, adGroupId_table, cmsSegId_table, cmsGroupId_table,
              finalGenderCode_table, ageLevel_table, pvalueLevel_table,
              shoppingLevel_table, occupation_table, newUserClassLevel_table,
              cateId_table, campaignId_table, customer_table, brand_table)
    idxs = (userId, adGroupId, cmsSegId, cmsGroupId, finalGenderCode,
            ageLevel, pvalueLevel, shoppingLevel, occupation,
            newUserClassLevel, cateId, campaignId, customer, brand)
    idxs = tuple(i.reshape(B) for i in idxs)

    flats = tuple(
        _make_detile(t.shape[0])(t.T).reshape(-1) for t in tables)

    emb = _sc_gather(*flats, *idxs)

    w1t = W1[:, :NT * D].T.reshape(NT, D, F1)
    w1p = W1[:, NT * D].reshape(1, F1)
    out = _mlp(emb, price.reshape(1, B), w1t, w1p, b1.reshape(1, F1),
               W2, b2.reshape(1, 1))
    return out.reshape(B, 1)


# speedup vs baseline: 1.1243x; 1.1243x over previous
import functools

import jax
import jax.numpy as jnp
from jax import lax
from jax.experimental import pallas as pl
from jax.experimental.pallas import tpu as pltpu
from jax.experimental.pallas import tpu_sc as plsc

B = 16384
D = 16
NT = 14
NH = 7
F1 = 128

NC = 2
NS = 16
NW = NC * NS
BPW = B // NW


@functools.lru_cache(maxsize=None)
def _make_detile(V):
    dbk = min(32768, ((V + 127) // 128) * 128)
    nb = (V + dbk - 1) // dbk
    NB = (V + 127) // 128

    def body(in_ref, out_ref):
        x = in_ref[...]
        out_ref[...] = x.reshape(D, dbk // 128, 128).transpose(1, 0, 2)

    return pl.pallas_call(
        body,
        grid=(nb,),
        in_specs=[pl.BlockSpec((D, dbk), lambda i: (0, i))],
        out_specs=pl.BlockSpec((dbk // 128, D, 128), lambda i: (i, 0, 0)),
        out_shape=jax.ShapeDtypeStruct((NB, D, 128), jnp.float32),
    )


_sc_mesh = plsc.VectorSubcoreMesh(core_axis_name="c", subcore_axis_name="s")


@functools.partial(
    pl.kernel,
    out_type=jax.ShapeDtypeStruct((NH * B * D,), jnp.float32),
    mesh=_sc_mesh,
    scratch_types=[
        pltpu.VMEM((NH * BPW,), jnp.int32),
        pltpu.VMEM((2 * 5 * BPW * D,), jnp.int32),
        pltpu.VMEM((5 * BPW * D,), jnp.float32),
        pltpu.SemaphoreType.DMA,
        pltpu.SemaphoreType.DMA,
    ],
    compiler_params=pltpu.CompilerParams(use_tc_tiling_on_sc=False),
)
def _sc_gather(*refs):
    tables = refs[:NH]
    idxs = refs[NH:2 * NH]
    out_hbm = refs[2 * NH]
    idx_v, addr_v, rows_v, sem, sem2 = refs[2 * NH + 1:]

    wid = lax.axis_index("s") * NC + lax.axis_index("c")
    base = wid * BPW

    idx_copies = [
        pltpu.async_copy(idxs[t].at[pl.ds(base, BPW)],
                         idx_v.at[pl.ds(t * BPW, BPW)], sem)
        for t in range(NH)
    ]
    for c in idx_copies:
        c.wait()

    cvec = lax.iota(jnp.int32, 16) * 128

    def _bcast(v, l):
        return lax.gather(
            v, jnp.full((16, 1), l, jnp.int32),
            dimension_numbers=lax.GatherDimensionNumbers(
                offset_dims=(), collapsed_slice_dims=(0,),
                start_index_map=(0,)),
            slice_sizes=(1,),
            mode=lax.GatherScatterMode.PROMISE_IN_BOUNDS)

    chunks = ((0, 1, 2, 3, 4), (5, 6))
    SLOT = 5 * BPW * D

    def build_addr(ts, slot):
        def addr_body(g, carry):
            for j, t in enumerate(ts):
                vr = idx_v[pl.ds(t * BPW + g * 16, 16)]
                vhi = ((vr >> 7) << 11) + (vr & 127)
                for l in range(16):
                    addr_v[pl.ds(slot * SLOT
                                 + (j * BPW + g * 16 + l) * D, D)] = (
                        _bcast(vhi, l) + cvec)
            return carry

        lax.fori_loop(0, BPW // 16, addr_body, 0)

    def fire_gathers(ts, slot):
        return [
            pltpu.async_copy(
                tables[t].at[addr_v.at[pl.ds(slot * SLOT + j * BPW * D,
                                             BPW * D)]],
                rows_v.at[pl.ds(j * BPW * D, BPW * D)], sem)
            for j, t in enumerate(ts)
        ]

    def fire_writes(ts):
        return [
            pltpu.async_copy(
                rows_v.at[pl.ds(j * BPW * D, BPW * D)],
                out_hbm.at[pl.ds((t * B + base) * D, BPW * D)], sem2)
            for j, t in enumerate(ts)
        ]

    build_addr(chunks[0], 0)
    gat = fire_gathers(chunks[0], 0)
    for k in range(1, len(chunks) + 1):
        if k < len(chunks):
            build_addr(chunks[k], k % 2)
        for c in gat:
            c.wait()
        wr = fire_writes(chunks[k - 1])
        for c in wr:
            c.wait()
        if k < len(chunks):
            gat = fire_gathers(chunks[k], k % 2)


BLK = 1024
MB = BLK * D // 128


def _mlp_body(emb_a_ref, emb_b_ref, price_ref, w1_ref, sp_ref, b1_ref,
              w2_ref, b2_ref, out_ref):
    acc = lax.dot_general(
        price_ref[...], sp_ref[...], (((1,), (0,)), ((), ())),
        preferred_element_type=jnp.float32,
        precision=lax.Precision.DEFAULT) + b1_ref[...]
    for t in range(NH):
        acc = acc + lax.dot_general(
            emb_a_ref[t], w1_ref[t], (((1,), (0,)), ((), ())),
            preferred_element_type=jnp.float32,
            precision=lax.Precision.DEFAULT)
    for t in range(NH):
        acc = acc + lax.dot_general(
            emb_b_ref[t], w1_ref[NH + t], (((1,), (0,)), ((), ())),
            preferred_element_type=jnp.float32,
            precision=lax.Precision.DEFAULT)
    acc = jnp.maximum(acc, 0.0)
    z = lax.dot_general(
        acc, w2_ref[...], (((1,), (0,)), ((), ())),
        preferred_element_type=jnp.float32,
        precision=lax.Precision.DEFAULT)
    z = z + b2_ref[...]
    out_ref[...] = 1.0 / (1.0 + jnp.exp(-z))


_mlp = pl.pallas_call(
    _mlp_body,
    grid=(B // BLK,),
    in_specs=[
        pl.BlockSpec((NH, MB, 128), lambda i: (0, i, 0)),
        pl.BlockSpec((NH, MB, 128), lambda i: (0, i, 0)),
        pl.BlockSpec((MB, 8), lambda i: (i, 0)),
        pl.BlockSpec((NT, 128, 8 * F1), lambda i: (0, 0, 0)),
        pl.BlockSpec((8, 8 * F1), lambda i: (0, 0)),
        pl.BlockSpec((1, 8 * F1), lambda i: (0, 0)),
        pl.BlockSpec((8 * F1, 8), lambda i: (0, 0)),
        pl.BlockSpec((1, 1), lambda i: (0, 0)),
    ],
    out_specs=pl.BlockSpec((MB, 8), lambda i: (i, 0)),
    out_shape=jax.ShapeDtypeStruct((B // 8, 8), jnp.float32),
)


def kernel(userId, cmsSegId, cmsGroupId, finalGenderCode, ageLevel,
           pvalueLevel, shoppingLevel, occupation, newUserClassLevel,
           adGroupId, cateId, campaignId, customer, brand, price,
           userId_table, cmsSegId_table, cmsGroupId_table,
           finalGenderCode_table, ageLevel_table, pvalueLevel_table,
           shoppingLevel_table, occupation_table, newUserClassLevel_table,
           adGroupId_table, cateId_table, campaignId_table, customer_table,
           brand_table, W1, b1, W2, b2):
    tables = (userId_table, adGroupId_table, cmsSegId_table, cmsGroupId_table,
              finalGenderCode_table, ageLevel_table, pvalueLevel_table,
              shoppingLevel_table, occupation_table, newUserClassLevel_table,
              cateId_table, campaignId_table, customer_table, brand_table)
    idxs = (userId, adGroupId, cmsSegId, cmsGroupId, finalGenderCode,
            ageLevel, pvalueLevel, shoppingLevel, occupation,
            newUserClassLevel, cateId, campaignId, customer, brand)
    idxs = tuple(i.reshape(B) for i in idxs)

    flats = [_make_detile(t.shape[0])(t.T).reshape(-1)
             for t in tables[:NH]]
    emb_a = _sc_gather(*flats, *idxs[:NH])
    flats_b = [_make_detile(t.shape[0])(t.T).reshape(-1)
               for t in tables[NH:]]
    emb_b = _sc_gather(*flats_b, *idxs[NH:])

    emb_a = emb_a.reshape(NH, B * D // 128, 128)
    emb_b = emb_b.reshape(NH, B * D // 128, 128)

    eye8 = jnp.eye(8, dtype=jnp.float32)
    w1t = W1[:, :NT * D].T.reshape(NT, D, F1)
    w1big = jnp.stack([jnp.kron(eye8, w1t[t]) for t in range(NT)])
    sprice = jnp.kron(eye8, W1[:, NT * D].reshape(1, F1))
    b1big = jnp.tile(b1.reshape(1, F1), (1, 8))
    w2big = jnp.kron(eye8, W2.T)

    out = _mlp(emb_a, emb_b, price.reshape(B // 8, 8), w1big, sprice, b1big,
               w2big, b2.reshape(1, 1))
    return out.reshape(B, 1)
